# baseline (speedup 1.0000x reference)
import jax
import jax.numpy as jnp
from jax import lax
from jax.experimental import pallas as pl
from jax.experimental.pallas import tpu as pltpu
from jax.experimental.pallas import tpu_sc as plsc

NUM_LAYERS = 32
DIM = 2048
TTL = 1024
HALF_LIFE = 256.0
LN2 = 0.6931471805599453
LANES = 16
NCHUNK = TTL // LANES
NDIM = DIM // LANES
BIG = 2 ** 30
CHUNK_ELEMS = LANES * DIM


def _sc_body(cs_hbm, ec_hbm, ev_hbm, bank_hbm, bs_hbm, bc_hbm,
             out_hbm,
             bs_v, bc_v, w_v, i_v, f_v, buf0, buf1, acc_v, ev_v,
             sem0, sem1):
    l = lax.axis_index("c") * 16 + lax.axis_index("s")
    pltpu.sync_copy(cs_hbm, i_v)
    pltpu.sync_copy(ec_hbm.at[l], f_v)
    pltpu.sync_copy(bs_hbm.at[l], bs_v)
    pltpu.sync_copy(bc_hbm.at[l], bc_v)

    def scan_body(i, carry):
        vsum, minkey, minempty, wpay = carry
        idx = lax.iota(jnp.int32, LANES) + i * LANES
        bs16 = bs_v[pl.ds(i * LANES, LANES)]
        bc16 = bc_v[pl.ds(i * LANES, LANES)]
        age = jnp.maximum(i_v[...] - bs16, 0).astype(jnp.float32)
        w16 = jnp.where(bs16 >= 0,
                        jnp.exp(age * (-LN2 / HALF_LIFE)) * bc16, 0.0)
        w_v[pl.ds(i * LANES, LANES)] = w16
        vsum = vsum + w16
        key = (bs16 + 2) * TTL + idx
        wpay = jnp.where(key < minkey, w16, wpay)
        minkey = jnp.minimum(minkey, key)
        minempty = jnp.minimum(minempty, jnp.where(bs16 == -1, idx, BIG))
        return vsum, minkey, minempty, wpay

    vsum, minkey, minempty, wpay = lax.fori_loop(
        0, NCHUNK, scan_body,
        (jnp.zeros((LANES,), jnp.float32),
         jnp.full((LANES,), BIG, jnp.int32),
         jnp.full((LANES,), BIG, jnp.int32),
         jnp.zeros((LANES,), jnp.float32)),
        unroll=4)

    ec = f_v[...][0]
    fe = minempty[0]
    mk = minkey[0]
    wp = wpay[0]
    tot = vsum[0]
    for t in range(1, LANES):
        fe = jnp.minimum(fe, minempty[t])
        take = minkey[t] < mk
        wp = jnp.where(take, wpay[t], wp)
        mk = jnp.minimum(mk, minkey[t])
        tot = tot + vsum[t]
    oldest = mk & (TTL - 1)
    has_empty = fe < BIG
    slot = jnp.where(has_empty, fe, oldest)
    w_slot = jnp.where(has_empty, 0.0, wp)
    ws = tot - w_slot + ec

    pltpu.async_copy(bank_hbm.at[l, pl.ds(0, LANES)], buf0, sem0)

    def chunk_compute(ci, buf):
        base_slot = ci * LANES
        idx = lax.iota(jnp.int32, LANES) + base_slot
        w16 = w_v[pl.ds(base_slot, LANES)]
        nz = jnp.minimum(jnp.abs(idx - slot), 1).astype(jnp.float32)
        wz16 = w16 * nz
        wt_vs = [jnp.full((LANES,), wz16[t], jnp.float32)
                 for t in range(LANES)]

        def dim_body(c, _):
            base = c * LANES
            a = acc_v[pl.ds(base, LANES)]
            for t in range(LANES):
                a = a + wt_vs[t] * buf[t, pl.ds(base, LANES)]
            acc_v[pl.ds(base, LANES)] = a
            return 0

        lax.fori_loop(0, NDIM, dim_body, 0)

    def zero_body(c, _):
        acc_v[pl.ds(c * LANES, LANES)] = jnp.zeros((LANES,), jnp.float32)
        return 0

    lax.fori_loop(0, NDIM, zero_body, 0)

    def ring_body(p, _):
        c0 = 2 * p
        c1 = 2 * p + 1
        pltpu.async_copy(bank_hbm.at[l, pl.ds(c1 * LANES, LANES)],
                         buf1, sem1)
        pltpu.make_async_copy(bank_hbm.at[l, pl.ds(0, LANES)],
                              buf0, sem0).wait()
        chunk_compute(c0, buf0)

        @pl.when(c1 + 1 < NCHUNK)
        def _():
            pltpu.async_copy(bank_hbm.at[l, pl.ds((c1 + 1) * LANES, LANES)],
                             buf0, sem0)

        pltpu.make_async_copy(bank_hbm.at[l, pl.ds(0, LANES)],
                              buf1, sem1).wait()
        chunk_compute(c1, buf1)
        return 0

    lax.fori_loop(0, NCHUNK // 2, ring_body, 0)

    pltpu.sync_copy(ev_hbm.at[l], ev_v)
    ec_v = jnp.full((LANES,), ec, jnp.float32)
    ws_v = jnp.full((LANES,), ws, jnp.float32)
    eps_v = jnp.full((LANES,), 1e-12, jnp.float32)
    denom = jnp.maximum(ws_v, eps_v)
    pos_v = jnp.sign(ws_v)

    def fin_body(c, _):
        base = c * LANES
        a = acc_v[pl.ds(base, LANES)] + ec_v * ev_v[pl.ds(base, LANES)]
        acc_v[pl.ds(base, LANES)] = a / denom * pos_v
        return 0

    lax.fori_loop(0, NDIM, fin_body, 0)
    pltpu.sync_copy(acc_v, out_hbm.at[l])


def kernel(evidence, event_counts, current_step, bank_evidence, bank_step,
           bank_event_count):
    cs = jnp.full((LANES,), current_step, dtype=jnp.int32)
    ec16 = jnp.broadcast_to(event_counts[:, None], (NUM_LAYERS, LANES))

    fn = pl.kernel(
        _sc_body,
        out_type=jax.ShapeDtypeStruct((NUM_LAYERS, DIM), jnp.float32),
        mesh=plsc.VectorSubcoreMesh(core_axis_name="c", subcore_axis_name="s"),
        scratch_types=[
            pltpu.VMEM((TTL,), jnp.int32),
            pltpu.VMEM((TTL,), jnp.float32),
            pltpu.VMEM((TTL,), jnp.float32),
            pltpu.VMEM((LANES,), jnp.int32),
            pltpu.VMEM((LANES,), jnp.float32),
            pltpu.VMEM((LANES, DIM), jnp.float32),
            pltpu.VMEM((LANES, DIM), jnp.float32),
            pltpu.VMEM((DIM,), jnp.float32),
            pltpu.VMEM((DIM,), jnp.float32),
            pltpu.SemaphoreType.DMA,
            pltpu.SemaphoreType.DMA,
        ],
    )
    return fn(cs, ec16, evidence, bank_evidence, bank_step, bank_event_count)


# speedup vs baseline: 1.2264x; 1.2264x over previous
import jax
import jax.numpy as jnp
from jax import lax
from jax.experimental import pallas as pl
from jax.experimental.pallas import tpu as pltpu
from jax.experimental.pallas import tpu_sc as plsc

NUM_LAYERS = 32
DIM = 2048
TTL = 1024
HALF_LIFE = 256.0
LN2 = 0.6931471805599453
LANES = 16
NCHUNK = TTL // LANES
NDIM = DIM // LANES
BIG = 2 ** 30
CHUNK_ELEMS = LANES * DIM


def _sc_body(cs_hbm, ec_hbm, ev_hbm, bank_hbm, bs_hbm, bc_hbm,
             out_hbm,
             bs_v, bc_v, w_v, i_v, f_v, buf0, buf1, acc_v, ev_v,
             sem0, sem1):
    l = lax.axis_index("c") * 16 + lax.axis_index("s")
    pltpu.sync_copy(cs_hbm, i_v)
    pltpu.sync_copy(ec_hbm.at[l], f_v)
    pltpu.sync_copy(bs_hbm.at[l], bs_v)
    pltpu.sync_copy(bc_hbm.at[l], bc_v)

    def scan_body(i, carry):
        vsum, minkey, minempty, wpay = carry
        idx = lax.iota(jnp.int32, LANES) + i * LANES
        bs16 = bs_v[pl.ds(i * LANES, LANES)]
        bc16 = bc_v[pl.ds(i * LANES, LANES)]
        age = jnp.maximum(i_v[...] - bs16, 0).astype(jnp.float32)
        w16 = jnp.where(bs16 >= 0,
                        jnp.exp(age * (-LN2 / HALF_LIFE)) * bc16, 0.0)
        w_v[pl.ds(i * LANES, LANES)] = w16
        vsum = vsum + w16
        key = (bs16 + 2) * TTL + idx
        wpay = jnp.where(key < minkey, w16, wpay)
        minkey = jnp.minimum(minkey, key)
        minempty = jnp.minimum(minempty, jnp.where(bs16 == -1, idx, BIG))
        return vsum, minkey, minempty, wpay

    vsum, minkey, minempty, wpay = lax.fori_loop(
        0, NCHUNK, scan_body,
        (jnp.zeros((LANES,), jnp.float32),
         jnp.full((LANES,), BIG, jnp.int32),
         jnp.full((LANES,), BIG, jnp.int32),
         jnp.zeros((LANES,), jnp.float32)),
        unroll=4)

    ec = f_v[...][0]
    fe = minempty[0]
    mk = minkey[0]
    wp = wpay[0]
    tot = vsum[0]
    for t in range(1, LANES):
        fe = jnp.minimum(fe, minempty[t])
        take = minkey[t] < mk
        wp = jnp.where(take, wpay[t], wp)
        mk = jnp.minimum(mk, minkey[t])
        tot = tot + vsum[t]
    oldest = mk & (TTL - 1)
    has_empty = fe < BIG
    slot = jnp.where(has_empty, fe, oldest)
    w_slot = jnp.where(has_empty, 0.0, wp)
    ws = tot - w_slot + ec

    pltpu.async_copy(bank_hbm.at[l, pl.ds(0, LANES)], buf0, sem0)

    def chunk_compute(ci, buf):
        base_slot = ci * LANES
        idx = lax.iota(jnp.int32, LANES) + base_slot
        w16 = w_v[pl.ds(base_slot, LANES)]
        nz = jnp.minimum(jnp.abs(idx - slot), 1).astype(jnp.float32)
        wz16 = w16 * nz
        wt_vs = [jnp.full((LANES,), wz16[t], jnp.float32)
                 for t in range(LANES)]

        def dim_body(c, _):
            base = c * LANES
            vals = [wt_vs[t] * buf[t, pl.ds(base, LANES)]
                    for t in range(LANES)]
            while len(vals) > 1:
                vals = [vals[i] + vals[i + 1]
                        for i in range(0, len(vals), 2)]
            acc_v[pl.ds(base, LANES)] = acc_v[pl.ds(base, LANES)] + vals[0]
            return 0

        lax.fori_loop(0, NDIM, dim_body, 0, unroll=2)

    def zero_body(c, _):
        acc_v[pl.ds(c * LANES, LANES)] = jnp.zeros((LANES,), jnp.float32)
        return 0

    lax.fori_loop(0, NDIM, zero_body, 0)

    def ring_body(p, _):
        c0 = 2 * p
        c1 = 2 * p + 1
        pltpu.async_copy(bank_hbm.at[l, pl.ds(c1 * LANES, LANES)],
                         buf1, sem1)
        pltpu.make_async_copy(bank_hbm.at[l, pl.ds(0, LANES)],
                              buf0, sem0).wait()
        chunk_compute(c0, buf0)

        @pl.when(c1 + 1 < NCHUNK)
        def _():
            pltpu.async_copy(bank_hbm.at[l, pl.ds((c1 + 1) * LANES, LANES)],
                             buf0, sem0)

        pltpu.make_async_copy(bank_hbm.at[l, pl.ds(0, LANES)],
                              buf1, sem1).wait()
        chunk_compute(c1, buf1)
        return 0

    lax.fori_loop(0, NCHUNK // 2, ring_body, 0)

    pltpu.sync_copy(ev_hbm.at[l], ev_v)
    ec_v = jnp.full((LANES,), ec, jnp.float32)
    ws_v = jnp.full((LANES,), ws, jnp.float32)
    eps_v = jnp.full((LANES,), 1e-12, jnp.float32)
    denom = jnp.maximum(ws_v, eps_v)
    pos_v = jnp.sign(ws_v)

    def fin_body(c, _):
        base = c * LANES
        a = acc_v[pl.ds(base, LANES)] + ec_v * ev_v[pl.ds(base, LANES)]
        acc_v[pl.ds(base, LANES)] = a / denom * pos_v
        return 0

    lax.fori_loop(0, NDIM, fin_body, 0)
    pltpu.sync_copy(acc_v, out_hbm.at[l])


def kernel(evidence, event_counts, current_step, bank_evidence, bank_step,
           bank_event_count):
    cs = jnp.full((LANES,), current_step, dtype=jnp.int32)
    ec16 = jnp.broadcast_to(event_counts[:, None], (NUM_LAYERS, LANES))

    fn = pl.kernel(
        _sc_body,
        out_type=jax.ShapeDtypeStruct((NUM_LAYERS, DIM), jnp.float32),
        mesh=plsc.VectorSubcoreMesh(core_axis_name="c", subcore_axis_name="s"),
        scratch_types=[
            pltpu.VMEM((TTL,), jnp.int32),
            pltpu.VMEM((TTL,), jnp.float32),
            pltpu.VMEM((TTL,), jnp.float32),
            pltpu.VMEM((LANES,), jnp.int32),
            pltpu.VMEM((LANES,), jnp.float32),
            pltpu.VMEM((LANES, DIM), jnp.float32),
            pltpu.VMEM((LANES, DIM), jnp.float32),
            pltpu.VMEM((DIM,), jnp.float32),
            pltpu.VMEM((DIM,), jnp.float32),
            pltpu.SemaphoreType.DMA,
            pltpu.SemaphoreType.DMA,
        ],
    )
    return fn(cs, ec16, evidence, bank_evidence, bank_step, bank_event_count)


# speedup vs baseline: 1.6497x; 1.3451x over previous
import jax
import jax.numpy as jnp
from jax import lax
from jax.experimental import pallas as pl
from jax.experimental.pallas import tpu as pltpu
from jax.experimental.pallas import tpu_sc as plsc

NUM_LAYERS = 32
DIM = 2048
TTL = 1024
HALF_LIFE = 256.0
LN2 = 0.6931471805599453
LANES = 16
NCHUNK = TTL // LANES
NDIM = DIM // LANES
BIG = 2 ** 30
CHUNK_ELEMS = LANES * DIM


def _sc_body(cs_hbm, ec_hbm, ev_hbm, bank_hbm, bs_hbm, bc_hbm,
             out_hbm,
             bs_v, bc_v, w_v, i_v, f_v, buf0, buf1, acc_v, ev_v,
             sem0, sem1):
    l = lax.axis_index("c") * 16 + lax.axis_index("s")
    pltpu.sync_copy(cs_hbm, i_v)
    pltpu.sync_copy(ec_hbm.at[l], f_v)
    pltpu.sync_copy(bs_hbm.at[l], bs_v)
    pltpu.sync_copy(bc_hbm.at[l], bc_v)

    def scan_body(i, carry):
        vsum, minkey, minempty, wpay = carry
        idx = lax.iota(jnp.int32, LANES) + i * LANES
        bs16 = bs_v[pl.ds(i * LANES, LANES)]
        bc16 = bc_v[pl.ds(i * LANES, LANES)]
        age = jnp.maximum(i_v[...] - bs16, 0).astype(jnp.float32)
        w16 = jnp.where(bs16 >= 0,
                        jnp.exp(age * (-LN2 / HALF_LIFE)) * bc16, 0.0)
        w_v[pl.ds(i * LANES, LANES)] = w16
        vsum = vsum + w16
        key = (bs16 + 2) * TTL + idx
        wpay = jnp.where(key < minkey, w16, wpay)
        minkey = jnp.minimum(minkey, key)
        minempty = jnp.minimum(minempty, jnp.where(bs16 == -1, idx, BIG))
        return vsum, minkey, minempty, wpay

    vsum, minkey, minempty, wpay = lax.fori_loop(
        0, NCHUNK, scan_body,
        (jnp.zeros((LANES,), jnp.float32),
         jnp.full((LANES,), BIG, jnp.int32),
         jnp.full((LANES,), BIG, jnp.int32),
         jnp.zeros((LANES,), jnp.float32)),
        unroll=4)

    ec = f_v[...][0]
    fe = minempty[0]
    mk = minkey[0]
    wp = wpay[0]
    tot = vsum[0]
    for t in range(1, LANES):
        fe = jnp.minimum(fe, minempty[t])
        take = minkey[t] < mk
        wp = jnp.where(take, wpay[t], wp)
        mk = jnp.minimum(mk, minkey[t])
        tot = tot + vsum[t]
    oldest = mk & (TTL - 1)
    has_empty = fe < BIG
    slot = jnp.where(has_empty, fe, oldest)
    w_slot = jnp.where(has_empty, 0.0, wp)
    ws = tot - w_slot + ec

    pltpu.async_copy(bank_hbm.at[l, pl.ds(0, LANES)], buf0, sem0)

    def chunk_compute(ci, buf):
        base_slot = ci * LANES
        idx = lax.iota(jnp.int32, LANES) + base_slot
        w16 = w_v[pl.ds(base_slot, LANES)]
        nz = jnp.minimum(jnp.abs(idx - slot), 1).astype(jnp.float32)
        wz16 = w16 * nz
        wt_vs = [jnp.full((LANES,), wz16[t], jnp.float32)
                 for t in range(LANES)]

        @plsc.parallel_loop(0, NDIM, step=1, unroll=4)
        def _dim_body(c):
            base = c * LANES
            vals = [wt_vs[t] * buf[t, pl.ds(base, LANES)]
                    for t in range(LANES)]
            while len(vals) > 1:
                vals = [vals[i] + vals[i + 1]
                        for i in range(0, len(vals), 2)]
            acc_v[pl.ds(base, LANES)] = acc_v[pl.ds(base, LANES)] + vals[0]

    def zero_body(c, _):
        acc_v[pl.ds(c * LANES, LANES)] = jnp.zeros((LANES,), jnp.float32)
        return 0

    lax.fori_loop(0, NDIM, zero_body, 0)

    def ring_body(p, _):
        c0 = 2 * p
        c1 = 2 * p + 1
        pltpu.async_copy(bank_hbm.at[l, pl.ds(c1 * LANES, LANES)],
                         buf1, sem1)
        pltpu.make_async_copy(bank_hbm.at[l, pl.ds(0, LANES)],
                              buf0, sem0).wait()
        chunk_compute(c0, buf0)

        @pl.when(c1 + 1 < NCHUNK)
        def _():
            pltpu.async_copy(bank_hbm.at[l, pl.ds((c1 + 1) * LANES, LANES)],
                             buf0, sem0)

        pltpu.make_async_copy(bank_hbm.at[l, pl.ds(0, LANES)],
                              buf1, sem1).wait()
        chunk_compute(c1, buf1)
        return 0

    lax.fori_loop(0, NCHUNK // 2, ring_body, 0)

    pltpu.sync_copy(ev_hbm.at[l], ev_v)
    ec_v = jnp.full((LANES,), ec, jnp.float32)
    ws_v = jnp.full((LANES,), ws, jnp.float32)
    eps_v = jnp.full((LANES,), 1e-12, jnp.float32)
    denom = jnp.maximum(ws_v, eps_v)
    pos_v = jnp.sign(ws_v)

    def fin_body(c, _):
        base = c * LANES
        a = acc_v[pl.ds(base, LANES)] + ec_v * ev_v[pl.ds(base, LANES)]
        acc_v[pl.ds(base, LANES)] = a / denom * pos_v
        return 0

    lax.fori_loop(0, NDIM, fin_body, 0)
    pltpu.sync_copy(acc_v, out_hbm.at[l])


def kernel(evidence, event_counts, current_step, bank_evidence, bank_step,
           bank_event_count):
    cs = jnp.full((LANES,), current_step, dtype=jnp.int32)
    ec16 = jnp.broadcast_to(event_counts[:, None], (NUM_LAYERS, LANES))

    fn = pl.kernel(
        _sc_body,
        out_type=jax.ShapeDtypeStruct((NUM_LAYERS, DIM), jnp.float32),
        mesh=plsc.VectorSubcoreMesh(core_axis_name="c", subcore_axis_name="s"),
        scratch_types=[
            pltpu.VMEM((TTL,), jnp.int32),
            pltpu.VMEM((TTL,), jnp.float32),
            pltpu.VMEM((TTL,), jnp.float32),
            pltpu.VMEM((LANES,), jnp.int32),
            pltpu.VMEM((LANES,), jnp.float32),
            pltpu.VMEM((LANES, DIM), jnp.float32),
            pltpu.VMEM((LANES, DIM), jnp.float32),
            pltpu.VMEM((DIM,), jnp.float32),
            pltpu.VMEM((DIM,), jnp.float32),
            pltpu.SemaphoreType.DMA,
            pltpu.SemaphoreType.DMA,
        ],
    )
    return fn(cs, ec16, evidence, bank_evidence, bank_step, bank_event_count)


# speedup vs baseline: 1.6530x; 1.0021x over previous
import jax
import jax.numpy as jnp
from jax import lax
from jax.experimental import pallas as pl
from jax.experimental.pallas import tpu as pltpu
from jax.experimental.pallas import tpu_sc as plsc

NUM_LAYERS = 32
DIM = 2048
TTL = 1024
HALF_LIFE = 256.0
LN2 = 0.6931471805599453
LANES = 16
NCHUNK = TTL // LANES
NDIM = DIM // LANES
BIG = 2 ** 30
CHUNK_ELEMS = LANES * DIM


def _sc_body(cs_hbm, ec_hbm, ev_hbm, bank_hbm, bs_hbm, bc_hbm,
             out_hbm,
             bs_v, bc_v, w_v, i_v, f_v, buf0, buf1, acc_v, ev_v,
             sem0, sem1):
    l = lax.axis_index("c") * 16 + lax.axis_index("s")
    pltpu.sync_copy(cs_hbm, i_v)
    pltpu.sync_copy(ec_hbm.at[l], f_v)
    pltpu.sync_copy(bs_hbm.at[l], bs_v)
    pltpu.sync_copy(bc_hbm.at[l], bc_v)

    def scan_body(i, carry):
        vsum, minkey, minempty, wpay = carry
        idx = lax.iota(jnp.int32, LANES) + i * LANES
        bs16 = bs_v[pl.ds(i * LANES, LANES)]
        bc16 = bc_v[pl.ds(i * LANES, LANES)]
        age = jnp.maximum(i_v[...] - bs16, 0).astype(jnp.float32)
        w16 = jnp.where(bs16 >= 0,
                        jnp.exp(age * (-LN2 / HALF_LIFE)) * bc16, 0.0)
        w_v[pl.ds(i * LANES, LANES)] = w16
        vsum = vsum + w16
        key = (bs16 + 2) * TTL + idx
        wpay = jnp.where(key < minkey, w16, wpay)
        minkey = jnp.minimum(minkey, key)
        minempty = jnp.minimum(minempty, jnp.where(bs16 == -1, idx, BIG))
        return vsum, minkey, minempty, wpay

    vsum, minkey, minempty, wpay = lax.fori_loop(
        0, NCHUNK, scan_body,
        (jnp.zeros((LANES,), jnp.float32),
         jnp.full((LANES,), BIG, jnp.int32),
         jnp.full((LANES,), BIG, jnp.int32),
         jnp.zeros((LANES,), jnp.float32)),
        unroll=4)

    ec = f_v[...][0]
    fe = minempty[0]
    mk = minkey[0]
    wp = wpay[0]
    tot = vsum[0]
    for t in range(1, LANES):
        fe = jnp.minimum(fe, minempty[t])
        take = minkey[t] < mk
        wp = jnp.where(take, wpay[t], wp)
        mk = jnp.minimum(mk, minkey[t])
        tot = tot + vsum[t]
    oldest = mk & (TTL - 1)
    has_empty = fe < BIG
    slot = jnp.where(has_empty, fe, oldest)
    w_slot = jnp.where(has_empty, 0.0, wp)
    ws = tot - w_slot + ec

    pltpu.async_copy(bank_hbm.at[l, pl.ds(0, LANES)], buf0, sem0)

    def chunk_compute(ci, buf):
        base_slot = ci * LANES
        idx = lax.iota(jnp.int32, LANES) + base_slot
        w16 = w_v[pl.ds(base_slot, LANES)]
        nz = jnp.minimum(jnp.abs(idx - slot), 1).astype(jnp.float32)
        wz16 = w16 * nz
        wt_vs = [jnp.full((LANES,), wz16[t], jnp.float32)
                 for t in range(LANES)]

        @plsc.parallel_loop(0, NDIM, step=1, unroll=8)
        def _dim_body(c):
            base = c * LANES
            vals = [wt_vs[t] * buf[t, pl.ds(base, LANES)]
                    for t in range(LANES)]
            while len(vals) > 1:
                vals = [vals[i] + vals[i + 1]
                        for i in range(0, len(vals), 2)]
            acc_v[pl.ds(base, LANES)] = acc_v[pl.ds(base, LANES)] + vals[0]

    def zero_body(c, _):
        acc_v[pl.ds(c * LANES, LANES)] = jnp.zeros((LANES,), jnp.float32)
        return 0

    lax.fori_loop(0, NDIM, zero_body, 0)

    def ring_body(p, _):
        c0 = 2 * p
        c1 = 2 * p + 1
        pltpu.async_copy(bank_hbm.at[l, pl.ds(c1 * LANES, LANES)],
                         buf1, sem1)
        pltpu.make_async_copy(bank_hbm.at[l, pl.ds(0, LANES)],
                              buf0, sem0).wait()
        chunk_compute(c0, buf0)

        @pl.when(c1 + 1 < NCHUNK)
        def _():
            pltpu.async_copy(bank_hbm.at[l, pl.ds((c1 + 1) * LANES, LANES)],
                             buf0, sem0)

        pltpu.make_async_copy(bank_hbm.at[l, pl.ds(0, LANES)],
                              buf1, sem1).wait()
        chunk_compute(c1, buf1)
        return 0

    lax.fori_loop(0, NCHUNK // 2, ring_body, 0)

    pltpu.sync_copy(ev_hbm.at[l], ev_v)
    ec_v = jnp.full((LANES,), ec, jnp.float32)
    ws_v = jnp.full((LANES,), ws, jnp.float32)
    eps_v = jnp.full((LANES,), 1e-12, jnp.float32)
    denom = jnp.maximum(ws_v, eps_v)
    pos_v = jnp.sign(ws_v)

    def fin_body(c, _):
        base = c * LANES
        a = acc_v[pl.ds(base, LANES)] + ec_v * ev_v[pl.ds(base, LANES)]
        acc_v[pl.ds(base, LANES)] = a / denom * pos_v
        return 0

    lax.fori_loop(0, NDIM, fin_body, 0)
    pltpu.sync_copy(acc_v, out_hbm.at[l])


def kernel(evidence, event_counts, current_step, bank_evidence, bank_step,
           bank_event_count):
    cs = jnp.full((LANES,), current_step, dtype=jnp.int32)
    ec16 = jnp.broadcast_to(event_counts[:, None], (NUM_LAYERS, LANES))

    fn = pl.kernel(
        _sc_body,
        out_type=jax.ShapeDtypeStruct((NUM_LAYERS, DIM), jnp.float32),
        mesh=plsc.VectorSubcoreMesh(core_axis_name="c", subcore_axis_name="s"),
        scratch_types=[
            pltpu.VMEM((TTL,), jnp.int32),
            pltpu.VMEM((TTL,), jnp.float32),
            pltpu.VMEM((TTL,), jnp.float32),
            pltpu.VMEM((LANES,), jnp.int32),
            pltpu.VMEM((LANES,), jnp.float32),
            pltpu.VMEM((LANES, DIM), jnp.float32),
            pltpu.VMEM((LANES, DIM), jnp.float32),
            pltpu.VMEM((DIM,), jnp.float32),
            pltpu.VMEM((DIM,), jnp.float32),
            pltpu.SemaphoreType.DMA,
            pltpu.SemaphoreType.DMA,
        ],
    )
    return fn(cs, ec16, evidence, bank_evidence, bank_step, bank_event_count)


# speedup vs baseline: 2.0721x; 1.2535x over previous
import jax
import jax.numpy as jnp
from jax import lax
from jax.experimental import pallas as pl
from jax.experimental.pallas import tpu as pltpu
from jax.experimental.pallas import tpu_sc as plsc

NUM_LAYERS = 32
DIM = 2048
TTL = 1024
HALF_LIFE = 256.0
LN2 = 0.6931471805599453
LANES = 16
NCHUNK = TTL // LANES
BIG = 2 ** 30


def _sc_weights(cs_hbm, bs_hbm, bc_hbm,
                w_hbm, vsum_hbm, minkey_hbm, minempty_hbm, wpay_hbm,
                bs_v, bc_v, w_v, i_v, f_v):
    l = lax.axis_index("c") * 16 + lax.axis_index("s")
    pltpu.sync_copy(cs_hbm, i_v)
    pltpu.sync_copy(bs_hbm.at[l], bs_v)
    pltpu.sync_copy(bc_hbm.at[l], bc_v)

    def body(i, carry):
        vsum, minkey, minempty, wpay = carry
        idx = lax.iota(jnp.int32, LANES) + i * LANES
        bs16 = bs_v[pl.ds(i * LANES, LANES)]
        bc16 = bc_v[pl.ds(i * LANES, LANES)]
        age = jnp.maximum(i_v[...] - bs16, 0).astype(jnp.float32)
        w16 = jnp.where(bs16 >= 0,
                        jnp.exp(age * (-LN2 / HALF_LIFE)) * bc16, 0.0)
        w_v[pl.ds(i * LANES, LANES)] = w16
        vsum = vsum + w16
        key = (bs16 + 2) * TTL + idx
        wpay = jnp.where(key < minkey, w16, wpay)
        minkey = jnp.minimum(minkey, key)
        minempty = jnp.minimum(minempty, jnp.where(bs16 == -1, idx, BIG))
        return vsum, minkey, minempty, wpay

    vsum, minkey, minempty, wpay = lax.fori_loop(
        0, NCHUNK, body,
        (jnp.zeros((LANES,), jnp.float32),
         jnp.full((LANES,), BIG, jnp.int32),
         jnp.full((LANES,), BIG, jnp.int32),
         jnp.zeros((LANES,), jnp.float32)))

    pltpu.sync_copy(w_v, w_hbm.at[l])
    f_v[...] = vsum
    pltpu.sync_copy(f_v, vsum_hbm.at[l])
    i_v[...] = minkey
    pltpu.sync_copy(i_v, minkey_hbm.at[l])
    i_v[...] = minempty
    pltpu.sync_copy(i_v, minempty_hbm.at[l])
    f_v[...] = wpay
    pltpu.sync_copy(f_v, wpay_hbm.at[l])


def _sc_weights_call(cs, bank_step, bank_event_count):
    fn = pl.kernel(
        _sc_weights,
        out_type=[
            jax.ShapeDtypeStruct((NUM_LAYERS, TTL), jnp.float32),
            jax.ShapeDtypeStruct((NUM_LAYERS, LANES), jnp.float32),
            jax.ShapeDtypeStruct((NUM_LAYERS, LANES), jnp.int32),
            jax.ShapeDtypeStruct((NUM_LAYERS, LANES), jnp.int32),
            jax.ShapeDtypeStruct((NUM_LAYERS, LANES), jnp.float32),
        ],
        mesh=plsc.VectorSubcoreMesh(core_axis_name="c", subcore_axis_name="s"),
        scratch_types=[
            pltpu.VMEM((TTL,), jnp.int32),
            pltpu.VMEM((TTL,), jnp.float32),
            pltpu.VMEM((TTL,), jnp.float32),
            pltpu.VMEM((LANES,), jnp.int32),
            pltpu.VMEM((LANES,), jnp.float32),
        ],
    )
    return fn(cs, bank_step, bank_event_count)


def _tc_body(ec_ref, vsum_ref, minkey_ref, minempty_ref, wpay_ref,
             w_ref, ev_ref, bank_ref, out_ref):
    l = pl.program_id(0)
    ec = ec_ref[l]

    minempty = minempty_ref[0]
    minkey = minkey_ref[0]
    first_empty = jnp.min(minempty)
    minkey_min = jnp.min(minkey)
    oldest = minkey_min & (TTL - 1)
    slot = jnp.where(first_empty < BIG, first_empty, oldest)
    w_oldest = jnp.sum(jnp.where(minkey == minkey_min, wpay_ref[0], 0.0))
    w_slot = jnp.where(first_empty < BIG, 0.0, w_oldest)
    ws = jnp.sum(vsum_ref[0]) - w_slot + ec

    w = w_ref[0]
    iota = lax.broadcasted_iota(jnp.int32, (1, TTL), 1)
    wz = jnp.where(iota == slot, 0.0, w)

    acc = jnp.dot(wz, bank_ref[0], preferred_element_type=jnp.float32)
    acc = acc + ec * ev_ref[0]
    res = acc / jnp.maximum(ws, 1e-12)
    out_ref[0] = jnp.where(ws > 0, res, jnp.zeros_like(res))


def kernel(evidence, event_counts, current_step, bank_evidence, bank_step,
           bank_event_count):
    cs = jnp.full((LANES,), current_step, dtype=jnp.int32)
    w_raw, vsum, minkey, minempty, wpay = _sc_weights_call(
        cs, bank_step, bank_event_count)

    w3 = w_raw.reshape(NUM_LAYERS, 1, TTL)
    ev3 = evidence.reshape(NUM_LAYERS, 1, DIM)
    vsum3 = vsum.reshape(NUM_LAYERS, 1, LANES)
    minkey3 = minkey.reshape(NUM_LAYERS, 1, LANES)
    minempty3 = minempty.reshape(NUM_LAYERS, 1, LANES)
    wpay3 = wpay.reshape(NUM_LAYERS, 1, LANES)

    lane_spec = pl.BlockSpec((1, 1, LANES), lambda l: (l, 0, 0))
    out = pl.pallas_call(
        _tc_body,
        grid=(NUM_LAYERS,),
        in_specs=[
            pl.BlockSpec(memory_space=pltpu.SMEM),
            lane_spec,
            lane_spec,
            lane_spec,
            lane_spec,
            pl.BlockSpec((1, 1, TTL), lambda l: (l, 0, 0)),
            pl.BlockSpec((1, 1, DIM), lambda l: (l, 0, 0)),
            pl.BlockSpec((1, TTL, DIM), lambda l: (l, 0, 0)),
        ],
        out_specs=pl.BlockSpec((1, 1, DIM), lambda l: (l, 0, 0)),
        out_shape=jax.ShapeDtypeStruct((NUM_LAYERS, 1, DIM), jnp.float32),
        compiler_params=pltpu.CompilerParams(
            dimension_semantics=("arbitrary",),
        ),
    )(event_counts, vsum3, minkey3, minempty3, wpay3, w3, ev3, bank_evidence)
    return out.reshape(NUM_LAYERS, DIM)
